# baseline (speedup 1.0000x reference)
import math

import jax
import jax.numpy as jnp
import numpy as np
from jax.experimental import pallas as pl

_B, _T, _H, _W, _C = 8, 16, 224, 224, 8
_F = _B * _T
_H1, _C1 = 56, 32
_H2, _C2 = 14, 128
_LIN_IN = _H2 * _H2 * _C2
_D = 512
_NF = _T // 2
_NSEG = 6
_EPS = 1e-5


def _gelu(v):
    return 0.5 * v * (1.0 + jax.lax.erf(v * (1.0 / math.sqrt(2.0))))


def _ln(v, g, b):
    mu = jnp.mean(v, axis=-1, keepdims=True)
    var = jnp.mean((v - mu) ** 2, axis=-1, keepdims=True)
    return (v - mu) * jax.lax.rsqrt(var + _EPS) * g + b


def _conv_mm(x_ref, w_ref, b_ref, g_ref, gb_ref, out_ref):
    a = jnp.dot(x_ref[...], w_ref[...], preferred_element_type=jnp.float32)
    a = a + b_ref[...]
    out_ref[...] = _gelu(_ln(a, g_ref[...], gb_ref[...]))


def _stage_b(a_ref, w_ref, b_ref, out_ref):
    k = pl.program_id(0)

    @pl.when(k == 0)
    def _():
        out_ref[...] = jnp.broadcast_to(b_ref[...], out_ref.shape)

    out_ref[...] += jnp.dot(a_ref[...], w_ref[...],
                            preferred_element_type=jnp.float32)


def _stage_c(h_ref, cre_ref, cim_ref, wg_ref, out_ref):
    lane = jax.lax.broadcasted_iota(jnp.int32, (1, 8), 1)
    for b in range(_B):
        hb = h_ref[b * _T:(b + 1) * _T, :]
        re = jnp.dot(cre_ref[...], hb, preferred_element_type=jnp.float32)
        im = jnp.dot(cim_ref[...], hb, preferred_element_type=jnp.float32)
        amp = jnp.sqrt(re * re + im * im)
        ampmean = jnp.mean(amp, axis=1, keepdims=True)
        logits = jnp.sum(ampmean * wg_ref[...], axis=0, keepdims=True)
        logits = jnp.where(lane < _NSEG, logits, -1e30)
        m1 = jnp.max(logits)
        i1 = jnp.argmax(logits, axis=1)[0]
        masked = jnp.where(lane == i1, -1e30, logits)
        m2 = jnp.max(masked)
        i2 = jnp.argmax(masked, axis=1)[0]
        e = jnp.exp(m2 - m1)
        gtop = 1.0 / (1.0 + e)
        gsec = e / (1.0 + e)
        row = jnp.where(lane == i1, gtop,
                        jnp.where(lane == i2, gsec, 0.0))
        out_ref[pl.ds(b, 1), :] = row


def _conv_call(xp, w, b, g, gb, blk_m):
    m, kdim = xp.shape
    n = w.shape[1]
    return pl.pallas_call(
        _conv_mm,
        grid=(m // blk_m,),
        in_specs=[
            pl.BlockSpec((blk_m, kdim), lambda i: (i, 0)),
            pl.BlockSpec((kdim, n), lambda i: (0, 0)),
            pl.BlockSpec((1, n), lambda i: (0, 0)),
            pl.BlockSpec((1, n), lambda i: (0, 0)),
            pl.BlockSpec((1, n), lambda i: (0, 0)),
        ],
        out_specs=pl.BlockSpec((blk_m, n), lambda i: (i, 0)),
        out_shape=jax.ShapeDtypeStruct((m, n), jnp.float32),
    )(xp, w, b.reshape(1, n), g.reshape(1, n), gb.reshape(1, n))


@jax.jit
def kernel(x, conv1_w, conv1_b, ln1_g, ln1_b, conv2_w, conv2_b, ln2_g,
           ln2_b, lin_w, lin_b, w_gate):
    xp = x.reshape(_F, _H1, 4, _H1, 4, _C).transpose(0, 1, 3, 2, 4, 5)
    xp = xp.reshape(_F * _H1 * _H1, 4 * 4 * _C)
    w1 = conv1_w.reshape(4 * 4 * _C, _C1)
    h1 = _conv_call(xp, w1, conv1_b, ln1_g, ln1_b, 3136)

    h1p = h1.reshape(_F, _H2, 4, _H2, 4, _C1).transpose(0, 1, 3, 2, 4, 5)
    h1p = h1p.reshape(_F * _H2 * _H2, 4 * 4 * _C1)
    w2 = conv2_w.reshape(4 * 4 * _C1, _C2)
    h2 = _conv_call(h1p, w2, conv2_b, ln2_g, ln2_b, 3136)

    featf = h2.reshape(_F, _LIN_IN)
    kb = 3584
    nk = _LIN_IN // kb
    h = pl.pallas_call(
        _stage_b,
        grid=(nk,),
        in_specs=[
            pl.BlockSpec((_F, kb), lambda k: (0, k)),
            pl.BlockSpec((kb, _D), lambda k: (k, 0)),
            pl.BlockSpec((1, _D), lambda k: (0, 0)),
        ],
        out_specs=pl.BlockSpec((_F, _D), lambda k: (0, 0)),
        out_shape=jax.ShapeDtypeStruct((_F, _D), jnp.float32),
    )(featf, lin_w, lin_b.reshape(1, _D))

    t = np.arange(_T)[:, None]
    f = np.arange(1, _NF + 1)[None, :]
    ang = 2.0 * np.pi * t * f / _T
    scale = 1.0 / np.sqrt(_T)
    cre = jnp.asarray((np.cos(ang) * scale).T, jnp.float32)
    cim = jnp.asarray((-np.sin(ang) * scale).T, jnp.float32)
    wg = jnp.pad(w_gate, ((0, 0), (0, 8 - _NSEG)))

    gates = pl.pallas_call(
        _stage_c,
        grid=(1,),
        in_specs=[
            pl.BlockSpec((_F, _D), lambda i: (0, 0)),
            pl.BlockSpec((_NF, _T), lambda i: (0, 0)),
            pl.BlockSpec((_NF, _T), lambda i: (0, 0)),
            pl.BlockSpec((_NF, 8), lambda i: (0, 0)),
        ],
        out_specs=pl.BlockSpec((_B, 8), lambda i: (0, 0)),
        out_shape=jax.ShapeDtypeStruct((_B, 8), jnp.float32),
    )(h, cre, cim, wg)

    return gates[:, :_NSEG]


# speedup vs baseline: 1.5187x; 1.5187x over previous
import math

import jax
import jax.numpy as jnp
import numpy as np
from jax.experimental import pallas as pl

_B, _T, _H, _W, _C = 8, 16, 224, 224, 8
_F = _B * _T
_H1, _C1 = 56, 32
_H2, _C2 = 14, 128
_LIN_IN = _H2 * _H2 * _C2
_D = 512
_NF = _T // 2
_NSEG = 6
_EPS = 1e-5


def _gelu(v):
    return 0.5 * v * (1.0 + jax.lax.erf(v * (1.0 / math.sqrt(2.0))))


def _ln(v, g, b):
    mu = jnp.mean(v, axis=-1, keepdims=True)
    var = jnp.mean((v - mu) ** 2, axis=-1, keepdims=True)
    return (v - mu) * jax.lax.rsqrt(var + _EPS) * g + b


def _stage_a(x_ref, w1_ref, b1t_ref, g1t_ref, gb1t_ref, m32_ref, w2_ref,
             b2_ref, g2_ref, gb2_ref, out_ref):
    xq = x_ref[0]
    xr = xq.reshape(_H1, 4, _H1, _C1)
    parts = [xr[:, ph].reshape(_H1 * _H1, _C1) for ph in range(4)]
    xcat = jnp.concatenate(parts, axis=1)
    araw = jnp.dot(xcat, w1_ref[...], preferred_element_type=jnp.float32)
    a3 = araw.reshape(_H1, _H1, _C1)

    m32 = m32_ref[...]
    acc = jnp.zeros((_H2 * _H2, _C2), jnp.float32) + b2_ref[...]
    for qh in range(4):
        aq = a3.reshape(_H2, 4, _H1, _C1)[:, qh]
        blocks = [aq.reshape(_H2, _H2, 4, _C1)[:, :, qw] for qw in range(4)]
        x2 = jnp.concatenate(blocks, axis=-1)
        x2 = x2.reshape(_H2 * _H2, 4 * _C1) + b1t_ref[...]
        mu = jnp.dot(x2, m32, preferred_element_type=jnp.float32)
        ex2 = jnp.dot(x2 * x2, m32, preferred_element_type=jnp.float32)
        var = ex2 - mu * mu
        y = (x2 - mu) * jax.lax.rsqrt(var + _EPS) * g1t_ref[...] + gb1t_ref[...]
        y = _gelu(y)
        acc = acc + jnp.dot(y, w2_ref[qh], preferred_element_type=jnp.float32)
    out_ref[0] = _gelu(_ln(acc, g2_ref[...], gb2_ref[...]))


def _stage_b(a_ref, w_ref, b_ref, out_ref):
    k = pl.program_id(0)

    @pl.when(k == 0)
    def _():
        out_ref[...] = jnp.broadcast_to(b_ref[...], out_ref.shape)

    out_ref[...] += jnp.dot(a_ref[...], w_ref[...],
                            preferred_element_type=jnp.float32)


def _stage_c(h_ref, cre_ref, cim_ref, wg_ref, out_ref):
    lane = jax.lax.broadcasted_iota(jnp.int32, (1, 8), 1)
    for b in range(_B):
        hb = h_ref[b * _T:(b + 1) * _T, :]
        re = jnp.dot(cre_ref[...], hb, preferred_element_type=jnp.float32)
        im = jnp.dot(cim_ref[...], hb, preferred_element_type=jnp.float32)
        amp = jnp.sqrt(re * re + im * im)
        ampmean = jnp.mean(amp, axis=1, keepdims=True)
        logits = jnp.sum(ampmean * wg_ref[...], axis=0, keepdims=True)
        logits = jnp.where(lane < _NSEG, logits, -1e30)
        m1 = jnp.max(logits)
        i1 = jnp.argmax(logits, axis=1)[0]
        masked = jnp.where(lane == i1, -1e30, logits)
        m2 = jnp.max(masked)
        i2 = jnp.argmax(masked, axis=1)[0]
        e = jnp.exp(m2 - m1)
        gtop = 1.0 / (1.0 + e)
        gsec = e / (1.0 + e)
        row = jnp.where(lane == i1, gtop,
                        jnp.where(lane == i2, gsec, 0.0))
        out_ref[pl.ds(b, 1), :] = row


@jax.jit
def kernel(x, conv1_w, conv1_b, ln1_g, ln1_b, conv2_w, conv2_b, ln2_g,
           ln2_b, lin_w, lin_b, w_gate):
    xv = x.reshape(_F, _H, _H1, _C1)
    w1 = conv1_w.reshape(4 * 4 * _C, _C1)
    w2 = conv2_w.reshape(4, 4 * _C1, _C2)
    tile4 = lambda v: jnp.tile(v, 4).reshape(1, 4 * _C1)
    m32 = jnp.asarray(np.kron(np.eye(4), np.ones((_C1, _C1)) / _C1),
                      jnp.float32)

    feat = pl.pallas_call(
        _stage_a,
        grid=(_F,),
        in_specs=[
            pl.BlockSpec((1, _H, _H1, _C1), lambda i: (i, 0, 0, 0)),
            pl.BlockSpec(w1.shape, lambda i: (0, 0)),
            pl.BlockSpec((1, _C2), lambda i: (0, 0)),
            pl.BlockSpec((1, _C2), lambda i: (0, 0)),
            pl.BlockSpec((1, _C2), lambda i: (0, 0)),
            pl.BlockSpec(m32.shape, lambda i: (0, 0)),
            pl.BlockSpec(w2.shape, lambda i: (0, 0, 0)),
            pl.BlockSpec((1, _C2), lambda i: (0, 0)),
            pl.BlockSpec((1, _C2), lambda i: (0, 0)),
            pl.BlockSpec((1, _C2), lambda i: (0, 0)),
        ],
        out_specs=pl.BlockSpec((1, _H2 * _H2, _C2), lambda i: (i, 0, 0)),
        out_shape=jax.ShapeDtypeStruct((_F, _H2 * _H2, _C2), jnp.float32),
    )(xv, w1, tile4(conv1_b), tile4(ln1_g), tile4(ln1_b), m32, w2,
      conv2_b.reshape(1, _C2), ln2_g.reshape(1, _C2), ln2_b.reshape(1, _C2))

    featf = feat.reshape(_F, _LIN_IN)
    kb = 3584
    nk = _LIN_IN // kb
    h = pl.pallas_call(
        _stage_b,
        grid=(nk,),
        in_specs=[
            pl.BlockSpec((_F, kb), lambda k: (0, k)),
            pl.BlockSpec((kb, _D), lambda k: (k, 0)),
            pl.BlockSpec((1, _D), lambda k: (0, 0)),
        ],
        out_specs=pl.BlockSpec((_F, _D), lambda k: (0, 0)),
        out_shape=jax.ShapeDtypeStruct((_F, _D), jnp.float32),
    )(featf, lin_w, lin_b.reshape(1, _D))

    t = np.arange(_T)[:, None]
    f = np.arange(1, _NF + 1)[None, :]
    ang = 2.0 * np.pi * t * f / _T
    scale = 1.0 / np.sqrt(_T)
    cre = jnp.asarray((np.cos(ang) * scale).T, jnp.float32)
    cim = jnp.asarray((-np.sin(ang) * scale).T, jnp.float32)
    wg = jnp.pad(w_gate, ((0, 0), (0, 8 - _NSEG)))

    gates = pl.pallas_call(
        _stage_c,
        grid=(1,),
        in_specs=[
            pl.BlockSpec((_F, _D), lambda i: (0, 0)),
            pl.BlockSpec((_NF, _T), lambda i: (0, 0)),
            pl.BlockSpec((_NF, _T), lambda i: (0, 0)),
            pl.BlockSpec((_NF, 8), lambda i: (0, 0)),
        ],
        out_specs=pl.BlockSpec((_B, 8), lambda i: (0, 0)),
        out_shape=jax.ShapeDtypeStruct((_B, 8), jnp.float32),
    )(h, cre, cim, wg)

    return gates[:, :_NSEG]
